# baseline (speedup 1.0000x reference)
import functools

import jax
import jax.numpy as jnp
from jax.experimental import pallas as pl
from jax.experimental.pallas import tpu as pltpu

H = 256
W = 256
N = 8192

PB = 2048
GC = 256
NCH = N // GC


def _raster_body(mx_ref, my_ref, opac_ref, neg_ref, rot_ref, s0_ref, s1_ref,
                 col_ref, img_ref, radii_ref, a_s, b_s, c_s, op_s):
    pid = pl.program_id(0)

    @pl.when(pid == 0)
    def _():
        theta = rot_ref[...] * (2.0 * jnp.pi)
        sx = s0_ref[...] * 0.02 + 1e-3
        sy = s1_ref[...] * 0.02 + 1e-3
        ct = jnp.cos(theta)
        st = jnp.sin(theta)
        a = ct * ct * sx * sx + st * st * sy * sy
        b = ct * st * (sx * sx - sy * sy)
        c = st * st * sx * sx + ct * ct * sy * sy
        det = a * c - b * b
        a_s[...] = c / det
        b_s[...] = -b / det
        c_s[...] = a / det
        op_s[...] = jnp.clip(opac_ref[...], 0.0, 0.99) * neg_ref[...]
        radii_ref[...] = jnp.ceil(
            3.0 * jnp.maximum(sx, sy) * float(max(H, W))).astype(jnp.int32)

    gidx = pid * PB + jax.lax.broadcasted_iota(jnp.int32, (PB, 1), 0)
    px = ((gidx % W).astype(jnp.float32) + 0.5) * (1.0 / W)
    py = ((gidx // W).astype(jnp.float32) + 0.5) * (1.0 / H)

    def chunk(j, acc):
        row = pl.ds(j, 1)
        mx = mx_ref[row, :]
        my = my_ref[row, :]
        A = a_s[row, :]
        B = b_s[row, :]
        C = c_s[row, :]
        op = op_s[row, :]
        colc = col_ref[pl.ds(j * GC, GC), :]
        dx = px - mx
        dy = py - my
        power = -0.5 * (A * dx * dx + 2.0 * B * dx * dy + C * dy * dy)
        G = jnp.where(power > -12.0, jnp.exp(power), 0.0)
        alpha = op * G
        return acc + jax.lax.dot_general(
            alpha, colc, (((1,), (0,)), ((), ())),
            preferred_element_type=jnp.float32)

    acc0 = jnp.zeros((PB, 3), dtype=jnp.float32)
    img_ref[...] = jax.lax.fori_loop(0, NCH, chunk, acc0)


def _as_rows(v):
    return v.reshape(NCH, GC)


@jax.jit
def kernel(means2D, opacities, colors, scale, rots, negative, bg):
    grid = (H * W) // PB
    full = pl.BlockSpec((NCH, GC), lambda i: (0, 0))
    img, radii = pl.pallas_call(
        _raster_body,
        grid=(grid,),
        in_specs=[full, full, full, full, full, full, full,
                  pl.BlockSpec((N, 3), lambda i: (0, 0))],
        out_specs=[
            pl.BlockSpec((PB, 3), lambda i: (i, 0)),
            pl.BlockSpec((NCH, GC), lambda i: (0, 0)),
        ],
        out_shape=[
            jax.ShapeDtypeStruct((H * W, 3), jnp.float32),
            jax.ShapeDtypeStruct((NCH, GC), jnp.int32),
        ],
        scratch_shapes=[pltpu.VMEM((NCH, GC), jnp.float32)] * 4,
    )(_as_rows(means2D[:, 0]), _as_rows(means2D[:, 1]),
      _as_rows(opacities[:, 0]), _as_rows(negative[:, 0]),
      _as_rows(rots[:, 0]), _as_rows(scale[:, 0]), _as_rows(scale[:, 1]),
      colors)

    color = img.T.reshape(3, H, W) + bg[:, None, None]
    return color, radii.reshape(N)
